# baseline (speedup 1.0000x reference)
import functools

import jax
import jax.numpy as jnp
from jax import lax
from jax.experimental import pallas as pl
from jax.experimental.pallas import tpu as pltpu
from jax.experimental.pallas import tpu_sc as plsc

_N = 1000012
_NP = 1000016
_B = 16384
_F = 26
_K = 16
_NIDX = _B * _F
_NC, _NS = 2, 16
_NW = _NC * _NS
_PER_W = _NIDX // _NW
_CH = 1664
_NSTEP = _PER_W // _CH

_R = 128


def _sc_fm(xf, et2, fc1):
    mesh = plsc.VectorSubcoreMesh(core_axis_name="c", subcore_axis_name="s")

    @functools.partial(
        pl.kernel,
        mesh=mesh,
        compiler_params=pltpu.CompilerParams(use_tc_tiling_on_sc=False),
        out_type=(
            jax.ShapeDtypeStruct((_K, _NIDX), jnp.float32),
            jax.ShapeDtypeStruct((_NIDX,), jnp.float32),
        ),
        scratch_types=[
            pltpu.VMEM((_CH,), jnp.int32),
            pltpu.VMEM((_K, _CH), jnp.float32),
            pltpu.VMEM((_CH,), jnp.float32),
        ],
    )
    def k(x_hbm, et_hbm, fc_hbm, e_out, f_out, idxb, ebuf, fbuf):
        wid = lax.axis_index("s") * _NC + lax.axis_index("c")
        base = wid * _PER_W
        for step in range(_NSTEP):
            j0 = base + step * _CH
            pltpu.sync_copy(x_hbm.at[pl.ds(j0, _CH)], idxb)
            for kk in range(_K):
                pltpu.sync_copy(et_hbm.at[kk].at[idxb], ebuf.at[kk])
            pltpu.sync_copy(fc_hbm.at[idxb], fbuf)
            pltpu.sync_copy(fbuf, f_out.at[pl.ds(j0, _CH)])
            pltpu.sync_copy(ebuf, e_out.at[:, pl.ds(j0, _CH)])

    return k(xf, et2, fc1)


def _red_body(e_ref, g_ref, fc_ref, w_ref, b_ref, o_ref):
    eb = e_ref[...]
    g = g_ref[...]
    s = jax.lax.dot(eb, g, precision=jax.lax.Precision.HIGHEST,
                    preferred_element_type=jnp.float32)
    ss = jax.lax.dot(eb * eb, g, precision=jax.lax.Precision.HIGHEST,
                     preferred_element_type=jnp.float32)
    inter = 0.5 * jnp.sum(s * s - ss, axis=0)
    fcs = jnp.sum(fc_ref[...], axis=1)
    z = fcs * w_ref[0, 0] + b_ref[0] + inter
    o_ref[...] = jax.nn.sigmoid(z)


def _tc_reduce(ekm, g, fc2, W, b):
    return pl.pallas_call(
        _red_body,
        grid=(_B // _R,),
        in_specs=[
            pl.BlockSpec((_K, _F * _R), lambda i: (0, i)),
            pl.BlockSpec((_F * _R, _R), lambda i: (0, 0)),
            pl.BlockSpec((_R, _F), lambda i: (i, 0)),
            pl.BlockSpec(memory_space=pltpu.SMEM),
            pl.BlockSpec(memory_space=pltpu.SMEM),
        ],
        out_specs=pl.BlockSpec((_R,), lambda i: (i,)),
        out_shape=jax.ShapeDtypeStruct((_B,), jnp.float32),
        compiler_params=pltpu.CompilerParams(
            dimension_semantics=("arbitrary",)),
    )(ekm, g, fc2, W, b)


def kernel(x, emb_table, fc_table, W, b):
    et2 = jnp.pad(emb_table, ((0, _NP - _N), (0, 0))).T
    fc1 = fc_table.reshape(_N)
    xf = x.reshape(_NIDX)
    ekm, fcv = _sc_fm(xf, et2, fc1)
    g = (lax.broadcasted_iota(jnp.int32, (_F * _R, _R), 0) // _F
         == lax.broadcasted_iota(jnp.int32, (_F * _R, _R), 1)
         ).astype(jnp.float32)
    return _tc_reduce(ekm, g, fcv.reshape(_B, _F), W, b)


# speedup vs baseline: 1.6625x; 1.6625x over previous
import functools

import jax
import jax.numpy as jnp
from jax import lax
from jax.experimental import pallas as pl
from jax.experimental.pallas import tpu as pltpu
from jax.experimental.pallas import tpu_sc as plsc

_N = 1000012
_NP = 1000016
_B = 16384
_F = 26
_K = 16
_NIDX = _B * _F
_NC, _NS = 2, 16
_NW = _NC * _NS
_RW = _B // _NW
_RCH = 64
_CH = _RCH * _F
_NSTEP = _RW // _RCH
_PER_W = _RW * _F


def _sc_fm(xp, et2, fc1, W, b):
    mesh = plsc.VectorSubcoreMesh(core_axis_name="c", subcore_axis_name="s")

    @functools.partial(
        pl.kernel,
        mesh=mesh,
        compiler_params=pltpu.CompilerParams(use_tc_tiling_on_sc=False),
        out_type=jax.ShapeDtypeStruct((_B,), jnp.float32),
        scratch_types=[
            pltpu.VMEM((_CH,), jnp.int32),
            pltpu.VMEM((_K, _CH), jnp.float32),
            pltpu.VMEM((_CH,), jnp.float32),
            pltpu.VMEM((_RCH,), jnp.float32),
            pltpu.VMEM((16,), jnp.float32),
            pltpu.VMEM((16,), jnp.float32),
            pltpu.SemaphoreType.DMA,
        ],
    )
    def k(x_hbm, et_hbm, fc_hbm, w_hbm, b_hbm, o_hbm,
          idxb, ebuf, fbuf, obuf, wvm, bvm, sem):
        pltpu.sync_copy(w_hbm, wvm)
        pltpu.sync_copy(b_hbm, bvm)
        w0 = wvm[...]
        b0 = bvm[...]
        wid = lax.axis_index("s") * _NC + lax.axis_index("c")
        base = wid * _PER_W
        rbase = wid * _RW
        for step in range(_NSTEP):
            j0 = base + step * _CH
            pltpu.sync_copy(x_hbm.at[pl.ds(j0, _CH)], idxb)
            cps = [pltpu.async_copy(et_hbm.at[kk].at[idxb], ebuf.at[kk], sem)
                   for kk in range(_K)]
            cps.append(pltpu.async_copy(fc_hbm.at[idxb], fbuf, sem))
            for cp in cps:
                cp.wait()

            @pl.loop(0, _RCH, step=16)
            def _(m):
                def kbody(kk, tacc):
                    s = ebuf[kk, pl.ds(m, 16)]
                    ss = s * s
                    for f in range(1, _F):
                        v = ebuf[kk, pl.ds(f * _RCH + m, 16)]
                        s = s + v
                        ss = ss + v * v
                    return tacc + s * s - ss

                t = lax.fori_loop(0, _K, kbody, jnp.zeros(16, jnp.float32))
                fcs = fbuf[pl.ds(m, 16)]
                for f in range(1, _F):
                    fcs = fcs + fbuf[pl.ds(f * _RCH + m, 16)]
                z = fcs * w0 + b0 + 0.5 * t
                obuf[pl.ds(m, 16)] = 1.0 / (1.0 + jnp.exp(-z))

            pltpu.sync_copy(obuf, o_hbm.at[pl.ds(rbase + step * _RCH, _RCH)])

    return k(xp, et2, fc1, W, b)


def kernel(x, emb_table, fc_table, W, b):
    zs = jnp.zeros((_NP - _N,), jnp.float32)
    pieces = []
    for kk in range(_K):
        pieces.append(emb_table[:, kk])
        pieces.append(zs)
    et2 = jnp.concatenate(pieces).reshape(_K, _NP)
    fc1 = fc_table.reshape(_N)
    xp = (x.reshape(_NW, _NSTEP, _RCH, _F)
          .transpose(0, 1, 3, 2)
          .reshape(_NIDX))
    w16 = jnp.broadcast_to(W.reshape(1), (16,))
    b16 = jnp.broadcast_to(b, (16,))
    return _sc_fm(xp, et2, fc1, w16, b16)


# speedup vs baseline: 5.5015x; 3.3091x over previous
import functools

import jax
import jax.numpy as jnp
from jax import lax
from jax.experimental import pallas as pl
from jax.experimental.pallas import tpu as pltpu
from jax.experimental.pallas import tpu_sc as plsc

_N = 1000012
_SP = 1000064
_B = 16384
_F = 26
_K = 16
_NIDX = _B * _F
_NC, _NS = 2, 16
_NW = _NC * _NS
_RW = _B // _NW
_RCH = 64
_CH = _RCH * _F
_NSTEP = _RW // _RCH
_PER_W = _RW * _F

_CC = 55552
_NBIG = _N // _CC
_REM = _N - _NBIG * _CC
_TPP = 20
_TPW = _K * _TPP // _NW


def _sc_relayout(emb_t, tailp):
    mesh = plsc.VectorSubcoreMesh(core_axis_name="c", subcore_axis_name="s")

    @functools.partial(
        pl.kernel,
        mesh=mesh,
        out_type=jax.ShapeDtypeStruct((_K * _SP,), jnp.float32),
        scratch_types=[
            pltpu.VMEM((_CC,), jnp.float32),
            pltpu.VMEM((128,), jnp.float32),
        ],
    )
    def k(et_hbm, tl_hbm, lin_hbm, buf, rbuf):
        wid = lax.axis_index("s") * _NC + lax.axis_index("c")
        for i in range(_TPW):
            t = wid * _TPW + i
            kk = t // _TPP
            sub = t % _TPP

            @pl.when(sub < _NBIG)
            def _():
                off = sub * _CC
                pltpu.sync_copy(et_hbm.at[kk].at[pl.ds(off, _CC)], buf)
                pltpu.sync_copy(buf, lin_hbm.at[pl.ds(kk * _SP + off, _CC)])

            @pl.when(sub == _NBIG)
            def _():
                off = _NBIG * _CC
                pltpu.sync_copy(tl_hbm.at[pl.ds(kk * 128, 128)], rbuf)
                pltpu.sync_copy(rbuf, lin_hbm.at[pl.ds(kk * _SP + off, 128)])

    return k(emb_t, tailp)


def _sc_fm(xp, et1, fc1, W, b):
    mesh = plsc.VectorSubcoreMesh(core_axis_name="c", subcore_axis_name="s")

    @functools.partial(
        pl.kernel,
        mesh=mesh,
        compiler_params=pltpu.CompilerParams(use_tc_tiling_on_sc=False),
        out_type=jax.ShapeDtypeStruct((_B,), jnp.float32),
        scratch_types=[
            pltpu.VMEM((_CH,), jnp.int32),
            pltpu.VMEM((_K, _CH), jnp.float32),
            pltpu.VMEM((_CH,), jnp.float32),
            pltpu.VMEM((_RCH,), jnp.float32),
            pltpu.VMEM((16,), jnp.float32),
            pltpu.VMEM((16,), jnp.float32),
            pltpu.SemaphoreType.DMA,
        ],
    )
    def k(x_hbm, et_hbm, fc_hbm, w_hbm, b_hbm, o_hbm,
          idxb, ebuf, fbuf, obuf, wvm, bvm, sem):
        pltpu.sync_copy(w_hbm, wvm)
        pltpu.sync_copy(b_hbm, bvm)
        w0 = wvm[...]
        b0 = bvm[...]
        wid = lax.axis_index("s") * _NC + lax.axis_index("c")
        base = wid * _PER_W
        rbase = wid * _RW
        for step in range(_NSTEP):
            j0 = base + step * _CH
            pltpu.sync_copy(x_hbm.at[pl.ds(j0, _CH)], idxb)
            cps = []
            for kk in range(_K):
                src = et_hbm.at[pl.ds(kk * _SP, _N)]
                cps.append(pltpu.async_copy(src.at[idxb], ebuf.at[kk], sem))
            cps.append(pltpu.async_copy(fc_hbm.at[idxb], fbuf, sem))
            for cp in cps:
                cp.wait()

            @pl.loop(0, _RCH, step=16)
            def _(m):
                def kbody(kk, tacc):
                    s = ebuf[kk, pl.ds(m, 16)]
                    ss = s * s
                    for f in range(1, _F):
                        v = ebuf[kk, pl.ds(f * _RCH + m, 16)]
                        s = s + v
                        ss = ss + v * v
                    return tacc + s * s - ss

                t = lax.fori_loop(0, _K, kbody, jnp.zeros(16, jnp.float32))
                fcs = fbuf[pl.ds(m, 16)]
                for f in range(1, _F):
                    fcs = fcs + fbuf[pl.ds(f * _RCH + m, 16)]
                z = fcs * w0 + b0 + 0.5 * t
                obuf[pl.ds(m, 16)] = 1.0 / (1.0 + jnp.exp(-z))

            pltpu.sync_copy(obuf, o_hbm.at[pl.ds(rbase + step * _RCH, _RCH)])

    return k(xp, et1, fc1, W, b)


def kernel(x, emb_table, fc_table, W, b):
    tail = emb_table[_NBIG * _CC:, :]
    tailp = jnp.pad(tail, ((0, 128 - _REM), (0, 0))).T.reshape(_K * 128)
    et1 = _sc_relayout(emb_table.T, tailp)
    fc1 = fc_table.reshape(_N)
    xp = (x.reshape(_NW, _NSTEP, _RCH, _F)
          .transpose(0, 1, 3, 2)
          .reshape(_NIDX))
    w16 = jnp.broadcast_to(W.reshape(1), (16,))
    b16 = jnp.broadcast_to(b, (16,))
    return _sc_fm(xp, et1, fc1, w16, b16)
